# baseline (speedup 1.0000x reference)
import functools

import jax
import jax.numpy as jnp
from jax import lax
from jax.experimental import pallas as pl
from jax.experimental.pallas import tpu as pltpu
from jax.experimental.pallas import tpu_sc as plsc

N = 10000
E = 320000
D = 128
NC = 2
NS = 16
NW = NC * NS
EDGES_PER_W = E // NW
CHUNK = 128
EDGES_PER_W_PAD = 10240
N_CHUNKS = EDGES_PER_W_PAD // CHUNK
NPAD = 10112
ROWS_PER_TILE = NPAD // NS


NBUF = 2
N_PHASES = 2
CHUNKS_PER_PHASE = N_CHUNKS // N_PHASES
N_GROUPS = CHUNKS_PER_PHASE // NBUF


def _seg_sum_body(h_hbm, idx_hbm, zero_hbm, s0_hbm, s1_hbm,
                  acc_shared, idx_v, rows_v, sems_g, sems_s):
    c = lax.axis_index("c")
    s = lax.axis_index("s")
    wid = s * NC + c

    pltpu.sync_copy(zero_hbm.at[pl.ds(s * ROWS_PER_TILE, ROWS_PER_TILE)],
                    acc_shared.at[pl.ds(s * ROWS_PER_TILE, ROWS_PER_TILE)])
    plsc.subcore_barrier()

    for p in range(N_PHASES):
        pltpu.sync_copy(idx_hbm.at[wid, p], idx_v)

        def group_step(g, carry):
            base = g * NBUF
            gathers = [
                pltpu.async_copy(h_hbm.at[idx_v.at[0, base + b]],
                                 rows_v.at[b], sems_g.at[b])
                for b in range(NBUF)
            ]
            scatters = []
            for b in range(NBUF):
                gathers[b].wait()
                scatters.append(
                    pltpu.async_copy(rows_v.at[b],
                                     acc_shared.at[idx_v.at[1, base + b]],
                                     sems_s.at[b], add=True))
            for sc in scatters:
                sc.wait()
            return carry

        lax.fori_loop(0, N_GROUPS, group_step, 0)
    plsc.subcore_barrier()

    rows = acc_shared.at[pl.ds(s * ROWS_PER_TILE, ROWS_PER_TILE)]

    @pl.when(c == 0)
    def _():
        pltpu.sync_copy(rows, s0_hbm.at[pl.ds(s * ROWS_PER_TILE, ROWS_PER_TILE)])

    @pl.when(c == 1)
    def _():
        pltpu.sync_copy(rows, s1_hbm.at[pl.ds(s * ROWS_PER_TILE, ROWS_PER_TILE)])


@jax.jit
def _seg_sum_sc(h, idx, zero):
    mesh = plsc.VectorSubcoreMesh(core_axis_name="c", subcore_axis_name="s")
    fn = pl.kernel(
        _seg_sum_body,
        mesh=mesh,
        out_type=(jax.ShapeDtypeStruct((NPAD, D), jnp.float32),
                  jax.ShapeDtypeStruct((NPAD, D), jnp.float32)),
        scratch_types=[
            pltpu.VMEM_SHARED((NPAD, D), jnp.float32),
            pltpu.VMEM((2, CHUNKS_PER_PHASE, CHUNK), jnp.int32),
            pltpu.VMEM((NBUF, CHUNK, D), jnp.float32),
            pltpu.SemaphoreType.DMA((NBUF,)),
            pltpu.SemaphoreType.DMA((NBUF,)),
        ],
    )
    return fn(h, idx, zero)


def _gru_cell(s0, s1, h, wg, wih, whh, bih, bhh):
    agg = jnp.dot(s0[...] + s1[...], wg[...],
                  preferred_element_type=jnp.float32)
    gi = jnp.dot(agg, wih[...], preferred_element_type=jnp.float32) + bih[...]
    gh = jnp.dot(h[...], whh[...], preferred_element_type=jnp.float32) + bhh[...]
    i_r, i_z, i_n = gi[:, :D], gi[:, D:2 * D], gi[:, 2 * D:]
    h_r, h_z, h_n = gh[:, :D], gh[:, D:2 * D], gh[:, 2 * D:]
    r = jax.nn.sigmoid(i_r + h_r)
    z = jax.nn.sigmoid(i_z + h_z)
    n = jnp.tanh(i_n + r * h_n)
    return (1.0 - z) * n + z * h[...]


def _gru_tc_body(s0, s1, h, wg, wih, whh, bih, bhh, out):
    out[...] = _gru_cell(s0, s1, h, wg, wih, whh, bih, bhh)


def _gru_mlp_tc_body(s0, s1, h, wg, wih, whh, bih, bhh, x, wh, wx, b, out):
    hn = _gru_cell(s0, s1, h, wg, wih, whh, bih, bhh)
    acc = jnp.dot(hn, wh[...], preferred_element_type=jnp.float32)
    acc += jnp.dot(x[...], wx[...], preferred_element_type=jnp.float32)
    out[...] = jnp.maximum(acc + b[...], 0.0)


BLK = 2000


def _gru_tc(s0, s1, h, wg, wihT, whhT, bih2, bhh2):
    grid = (N // BLK,)
    row = lambda i: (i, 0)
    fix = lambda i: (0, 0)
    return pl.pallas_call(
        _gru_tc_body,
        grid=grid,
        in_specs=[
            pl.BlockSpec((BLK, D), row),
            pl.BlockSpec((BLK, D), row),
            pl.BlockSpec((BLK, D), row),
            pl.BlockSpec((D, D), fix),
            pl.BlockSpec((D, 3 * D), fix),
            pl.BlockSpec((D, 3 * D), fix),
            pl.BlockSpec((1, 3 * D), fix),
            pl.BlockSpec((1, 3 * D), fix),
        ],
        out_specs=pl.BlockSpec((BLK, D), row),
        out_shape=jax.ShapeDtypeStruct((NPAD, D), jnp.float32),
    )(s0, s1, h, wg, wihT, whhT, bih2, bhh2)


def _gru_mlp_tc(s0, s1, h, wg, wihT, whhT, bih2, bhh2, x, whT, wxT, b2):
    grid = (N // BLK,)
    row = lambda i: (i, 0)
    fix = lambda i: (0, 0)
    return pl.pallas_call(
        _gru_mlp_tc_body,
        grid=grid,
        in_specs=[
            pl.BlockSpec((BLK, D), row),
            pl.BlockSpec((BLK, D), row),
            pl.BlockSpec((BLK, D), row),
            pl.BlockSpec((D, D), fix),
            pl.BlockSpec((D, 3 * D), fix),
            pl.BlockSpec((D, 3 * D), fix),
            pl.BlockSpec((1, 3 * D), fix),
            pl.BlockSpec((1, 3 * D), fix),
            pl.BlockSpec((BLK, D), row),
            pl.BlockSpec((D, D), fix),
            pl.BlockSpec((D, D), fix),
            pl.BlockSpec((1, D), fix),
        ],
        out_specs=pl.BlockSpec((BLK, D), row),
        out_shape=jax.ShapeDtypeStruct((N, D), jnp.float32),
    )(s0, s1, h, wg, wihT, whhT, bih2, bhh2, x, whT, wxT, b2)


def kernel(x, edge_index, ggnn_weight, W_ih, W_hh, b_ih, b_hh, W_out, b_out):
    src = edge_index[0]
    dst = edge_index[1]
    zero = jnp.zeros((NPAD, D), jnp.float32)
    wihT = W_ih.T
    whhT = W_hh.T
    bih2 = b_ih.reshape(1, 3 * D)
    bhh2 = b_hh.reshape(1, 3 * D)
    whT = W_out[:, :D].T
    wxT = W_out[:, D:].T
    b2 = b_out.reshape(1, -1)

    pad_w = EDGES_PER_W_PAD - EDGES_PER_W
    pad_src = (jnp.arange(NW * pad_w, dtype=jnp.int32) % N).reshape(NW, pad_w)
    pad_dst = (N + jnp.arange(NW * pad_w, dtype=jnp.int32) %
               (NPAD - N)).reshape(NW, pad_w)
    src2 = jnp.concatenate([src.reshape(NW, EDGES_PER_W), pad_src], axis=1)
    dst2 = jnp.concatenate([dst.reshape(NW, EDGES_PER_W), pad_dst], axis=1)
    src4 = src2.reshape(NW, N_PHASES, 1, CHUNKS_PER_PHASE, CHUNK)
    dst4 = dst2.reshape(NW, N_PHASES, 1, CHUNKS_PER_PHASE, CHUNK)
    idx = jnp.concatenate([src4, dst4], axis=2)
    h = jnp.pad(x, ((0, NPAD - N), (0, 0)))
    for i in range(2):
        s0, s1 = _seg_sum_sc(h, idx, zero)
        h = _gru_tc(s0, s1, h, ggnn_weight[i], wihT, whhT, bih2, bhh2)
    s0, s1 = _seg_sum_sc(h, idx, zero)
    return _gru_mlp_tc(s0, s1, h, ggnn_weight[2], wihT, whhT, bih2, bhh2,
                       x, whT, wxT, b2)


# speedup vs baseline: 1.1172x; 1.1172x over previous
import functools

import jax
import jax.numpy as jnp
from jax import lax
from jax.experimental import pallas as pl
from jax.experimental.pallas import tpu as pltpu
from jax.experimental.pallas import tpu_sc as plsc

N = 10000
E = 320000
D = 128
NC = 2
NS = 16
NW = NC * NS
EDGES_PER_W = E // NW
CHUNK = 64
EDGES_PER_W_PAD = 10240
N_CHUNKS = EDGES_PER_W_PAD // CHUNK
NPAD = 10112
ROWS_PER_TILE = NPAD // NS


NBUF = 4
N_PHASES = 4
CHUNKS_PER_PHASE = N_CHUNKS // N_PHASES
PAIR = 2


def _seg_sum_body(h_hbm, idx_hbm, zero_hbm, s0_hbm, s1_hbm,
                  acc_shared, idx_v, rows_v, sems_g, sems_s):
    c = lax.axis_index("c")
    s = lax.axis_index("s")
    wid = s * NC + c

    pltpu.sync_copy(zero_hbm.at[pl.ds(s * ROWS_PER_TILE, ROWS_PER_TILE)],
                    acc_shared.at[pl.ds(s * ROWS_PER_TILE, ROWS_PER_TILE)])
    plsc.subcore_barrier()

    def issue_gather(c):
        return pltpu.async_copy(h_hbm.at[idx_v.at[0, c]],
                                rows_v.at[c % NBUF], sems_g.at[c % NBUF])

    def issue_scatter(c):
        return pltpu.async_copy(rows_v.at[c % NBUF],
                                acc_shared.at[idx_v.at[1, c]],
                                sems_s.at[c % NBUF], add=True)

    for p in range(N_PHASES):
        pltpu.sync_copy(idx_hbm.at[wid, p], idx_v)

        gh = {c: issue_gather(c) for c in range(PAIR)}
        for c in range(0, CHUNKS_PER_PHASE, PAIR):
            for d in range(PAIR):
                gh.pop(c + d).wait()
            if c + PAIR < CHUNKS_PER_PHASE:
                for d in range(PAIR):
                    gh[c + PAIR + d] = issue_gather(c + PAIR + d)
            scatters = [issue_scatter(c + d) for d in range(PAIR)]
            for sc in scatters:
                sc.wait()
    plsc.subcore_barrier()

    rows = acc_shared.at[pl.ds(s * ROWS_PER_TILE, ROWS_PER_TILE)]

    @pl.when(c == 0)
    def _():
        pltpu.sync_copy(rows, s0_hbm.at[pl.ds(s * ROWS_PER_TILE, ROWS_PER_TILE)])

    @pl.when(c == 1)
    def _():
        pltpu.sync_copy(rows, s1_hbm.at[pl.ds(s * ROWS_PER_TILE, ROWS_PER_TILE)])


@jax.jit
def _seg_sum_sc(h, idx, zero):
    mesh = plsc.VectorSubcoreMesh(core_axis_name="c", subcore_axis_name="s")
    fn = pl.kernel(
        _seg_sum_body,
        mesh=mesh,
        out_type=(jax.ShapeDtypeStruct((NPAD, D), jnp.float32),
                  jax.ShapeDtypeStruct((NPAD, D), jnp.float32)),
        scratch_types=[
            pltpu.VMEM_SHARED((NPAD, D), jnp.float32),
            pltpu.VMEM((2, CHUNKS_PER_PHASE, CHUNK), jnp.int32),
            pltpu.VMEM((NBUF, CHUNK, D), jnp.float32),
            pltpu.SemaphoreType.DMA((NBUF,)),
            pltpu.SemaphoreType.DMA((NBUF,)),
        ],
    )
    return fn(h, idx, zero)


def _gru_cell(s0, s1, h, wg, wih, whh, bih, bhh):
    agg = jnp.dot(s0[...] + s1[...], wg[...],
                  preferred_element_type=jnp.float32)
    gi = jnp.dot(agg, wih[...], preferred_element_type=jnp.float32) + bih[...]
    gh = jnp.dot(h[...], whh[...], preferred_element_type=jnp.float32) + bhh[...]
    i_r, i_z, i_n = gi[:, :D], gi[:, D:2 * D], gi[:, 2 * D:]
    h_r, h_z, h_n = gh[:, :D], gh[:, D:2 * D], gh[:, 2 * D:]
    r = jax.nn.sigmoid(i_r + h_r)
    z = jax.nn.sigmoid(i_z + h_z)
    n = jnp.tanh(i_n + r * h_n)
    return (1.0 - z) * n + z * h[...]


def _gru_tc_body(s0, s1, h, wg, wih, whh, bih, bhh, out):
    out[...] = _gru_cell(s0, s1, h, wg, wih, whh, bih, bhh)


def _gru_mlp_tc_body(s0, s1, h, wg, wih, whh, bih, bhh, x, wh, wx, b, out):
    hn = _gru_cell(s0, s1, h, wg, wih, whh, bih, bhh)
    acc = jnp.dot(hn, wh[...], preferred_element_type=jnp.float32)
    acc += jnp.dot(x[...], wx[...], preferred_element_type=jnp.float32)
    out[...] = jnp.maximum(acc + b[...], 0.0)


BLK = 2000


def _gru_tc(s0, s1, h, wg, wihT, whhT, bih2, bhh2):
    grid = (N // BLK,)
    row = lambda i: (i, 0)
    fix = lambda i: (0, 0)
    return pl.pallas_call(
        _gru_tc_body,
        grid=grid,
        in_specs=[
            pl.BlockSpec((BLK, D), row),
            pl.BlockSpec((BLK, D), row),
            pl.BlockSpec((BLK, D), row),
            pl.BlockSpec((D, D), fix),
            pl.BlockSpec((D, 3 * D), fix),
            pl.BlockSpec((D, 3 * D), fix),
            pl.BlockSpec((1, 3 * D), fix),
            pl.BlockSpec((1, 3 * D), fix),
        ],
        out_specs=pl.BlockSpec((BLK, D), row),
        out_shape=jax.ShapeDtypeStruct((NPAD, D), jnp.float32),
    )(s0, s1, h, wg, wihT, whhT, bih2, bhh2)


def _gru_mlp_tc(s0, s1, h, wg, wihT, whhT, bih2, bhh2, x, whT, wxT, b2):
    grid = (N // BLK,)
    row = lambda i: (i, 0)
    fix = lambda i: (0, 0)
    return pl.pallas_call(
        _gru_mlp_tc_body,
        grid=grid,
        in_specs=[
            pl.BlockSpec((BLK, D), row),
            pl.BlockSpec((BLK, D), row),
            pl.BlockSpec((BLK, D), row),
            pl.BlockSpec((D, D), fix),
            pl.BlockSpec((D, 3 * D), fix),
            pl.BlockSpec((D, 3 * D), fix),
            pl.BlockSpec((1, 3 * D), fix),
            pl.BlockSpec((1, 3 * D), fix),
            pl.BlockSpec((BLK, D), row),
            pl.BlockSpec((D, D), fix),
            pl.BlockSpec((D, D), fix),
            pl.BlockSpec((1, D), fix),
        ],
        out_specs=pl.BlockSpec((BLK, D), row),
        out_shape=jax.ShapeDtypeStruct((N, D), jnp.float32),
    )(s0, s1, h, wg, wihT, whhT, bih2, bhh2, x, whT, wxT, b2)


def kernel(x, edge_index, ggnn_weight, W_ih, W_hh, b_ih, b_hh, W_out, b_out):
    src = edge_index[0]
    dst = edge_index[1]
    zero = jnp.zeros((NPAD, D), jnp.float32)
    wihT = W_ih.T
    whhT = W_hh.T
    bih2 = b_ih.reshape(1, 3 * D)
    bhh2 = b_hh.reshape(1, 3 * D)
    whT = W_out[:, :D].T
    wxT = W_out[:, D:].T
    b2 = b_out.reshape(1, -1)

    pad_w = EDGES_PER_W_PAD - EDGES_PER_W
    pad_src = (jnp.arange(NW * pad_w, dtype=jnp.int32) % N).reshape(NW, pad_w)
    pad_dst = (N + jnp.arange(NW * pad_w, dtype=jnp.int32) %
               (NPAD - N)).reshape(NW, pad_w)
    src2 = jnp.concatenate([src.reshape(NW, EDGES_PER_W), pad_src], axis=1)
    dst2 = jnp.concatenate([dst.reshape(NW, EDGES_PER_W), pad_dst], axis=1)
    src4 = src2.reshape(NW, N_PHASES, 1, CHUNKS_PER_PHASE, CHUNK)
    dst4 = dst2.reshape(NW, N_PHASES, 1, CHUNKS_PER_PHASE, CHUNK)
    idx = jnp.concatenate([src4, dst4], axis=2)
    h = jnp.pad(x, ((0, NPAD - N), (0, 0)))
    for i in range(2):
        s0, s1 = _seg_sum_sc(h, idx, zero)
        h = _gru_tc(s0, s1, h, ggnn_weight[i], wihT, whhT, bih2, bhh2)
    s0, s1 = _seg_sum_sc(h, idx, zero)
    return _gru_mlp_tc(s0, s1, h, ggnn_weight[2], wihT, whhT, bih2, bhh2,
                       x, whT, wxT, b2)
